# baseline (speedup 1.0000x reference)
import jax
import jax.numpy as jnp
from jax.experimental import pallas as pl
from jax.experimental.pallas import tpu as pltpu

_DIM = 128
_NH = 8
_CHUNK = 5000
_NBUF = 8
_NSLICE = 4


def _make_body(nbuf):
    def _lsh_body(x_hbm, rv_ref, o_ref, buf, sem):
        i = pl.program_id(0)
        nsteps = pl.num_programs(0)
        slot = jax.lax.rem(i, nbuf)

        @pl.when(i == 0)
        def _prefetch():
            for k in range(nbuf):
                pltpu.make_async_copy(
                    x_hbm.at[pl.ds(k * _CHUNK, _CHUNK), :],
                    buf.at[k],
                    sem.at[k],
                ).start()

        pltpu.make_async_copy(
            x_hbm.at[pl.ds(i * _CHUNK, _CHUNK), :],
            buf.at[slot],
            sem.at[slot],
        ).wait()

        proj_t = jax.lax.dot_general(
            rv_ref[...], buf[slot],
            dimension_numbers=(((0,), (1,)), ((), ())),
            preferred_element_type=jnp.float32,
        )
        buckets = jnp.floor(proj_t).astype(jnp.int32) & 1023
        o_ref[...] = buckets.astype(jnp.float32)[None]

        @pl.when(i + nbuf < nsteps)
        def _next():
            pltpu.make_async_copy(
                x_hbm.at[pl.ds((i + nbuf) * _CHUNK, _CHUNK), :],
                buf.at[slot],
                sem.at[slot],
            ).start()

    return _lsh_body


def _lsh_slice(x, random_vectors):
    n = x.shape[0]
    nsteps = n // _CHUNK
    nbuf = min(_NBUF, nsteps)
    out_t = pl.pallas_call(
        _make_body(nbuf),
        grid=(nsteps,),
        in_specs=[
            pl.BlockSpec(memory_space=pltpu.MemorySpace.HBM),
            pl.BlockSpec((_DIM, _NH), lambda i: (0, 0)),
        ],
        out_specs=pl.BlockSpec((1, _NH, _CHUNK), lambda i: (i, 0, 0)),
        out_shape=jax.ShapeDtypeStruct((nsteps, _NH, _CHUNK), jnp.float32),
        scratch_shapes=[
            pltpu.VMEM((nbuf, _CHUNK, _DIM), jnp.float32),
            pltpu.SemaphoreType.DMA((nbuf,)),
        ],
        compiler_params=pltpu.CompilerParams(
            dimension_semantics=("arbitrary",),
        ),
    )(x, random_vectors)
    return jnp.swapaxes(out_t, 1, 2).reshape(n, _NH)


def kernel(x, random_vectors):
    n = x.shape[0]
    rows = n // _NSLICE
    if rows == 0 or rows % _CHUNK != 0:
        return _lsh_slice(x, random_vectors)
    parts = [
        _lsh_slice(jax.lax.slice_in_dim(x, s * rows, (s + 1) * rows), random_vectors)
        for s in range(_NSLICE)
    ]
    return jnp.concatenate(parts, axis=0)


# speedup vs baseline: 2.1492x; 2.1492x over previous
import jax
import jax.numpy as jnp
from jax.experimental import pallas as pl
from jax.experimental.pallas import tpu as pltpu

_DIM = 128
_NH = 8
_CHUNK = 5000
_NBUF = 8


def _make_body(nbuf):
    def _lsh_body(x_hbm, rv_ref, o_ref, buf, sem):
        i = pl.program_id(0)
        nsteps = pl.num_programs(0)
        slot = jax.lax.rem(i, nbuf)

        @pl.when(i == 0)
        def _prefetch():
            for k in range(nbuf):
                pltpu.make_async_copy(
                    x_hbm.at[pl.ds(k * _CHUNK, _CHUNK), :],
                    buf.at[k],
                    sem.at[k],
                ).start()

        pltpu.make_async_copy(
            x_hbm.at[pl.ds(i * _CHUNK, _CHUNK), :],
            buf.at[slot],
            sem.at[slot],
        ).wait()

        proj_t = jax.lax.dot_general(
            rv_ref[...], buf[slot],
            dimension_numbers=(((0,), (1,)), ((), ())),
            preferred_element_type=jnp.float32,
        )
        buckets = jnp.floor(proj_t).astype(jnp.int32) & 1023
        o_ref[...] = buckets.astype(jnp.float32)[None]

        @pl.when(i + nbuf < nsteps)
        def _next():
            pltpu.make_async_copy(
                x_hbm.at[pl.ds((i + nbuf) * _CHUNK, _CHUNK), :],
                buf.at[slot],
                sem.at[slot],
            ).start()

    return _lsh_body


def _lsh_slice(x, random_vectors):
    n = x.shape[0]
    nsteps = n // _CHUNK
    nbuf = min(_NBUF, nsteps)
    out_t = pl.pallas_call(
        _make_body(nbuf),
        grid=(nsteps,),
        in_specs=[
            pl.BlockSpec(memory_space=pltpu.MemorySpace.HBM),
            pl.BlockSpec((_DIM, _NH), lambda i: (0, 0)),
        ],
        out_specs=pl.BlockSpec((1, _NH, _CHUNK), lambda i: (i, 0, 0)),
        out_shape=jax.ShapeDtypeStruct((nsteps, _NH, _CHUNK), jnp.float32),
        scratch_shapes=[
            pltpu.VMEM((nbuf, _CHUNK, _DIM), jnp.float32),
            pltpu.SemaphoreType.DMA((nbuf,)),
        ],
        compiler_params=pltpu.CompilerParams(
            dimension_semantics=("arbitrary",),
        ),
    )(x, random_vectors)
    return jnp.swapaxes(out_t, 1, 2).reshape(n, _NH)


def kernel(x, random_vectors):
    return _lsh_slice(x, random_vectors)
